# baseline (speedup 1.0000x reference)
import functools

import jax
import jax.numpy as jnp
from jax import lax
from jax.experimental import pallas as pl
from jax.experimental.pallas import tpu as pltpu
from jax.experimental.pallas import tpu_sc as plsc

D = 64
HALF = 32
LANES = 16
QROWS = 128

_DNUMS = lax.GatherDimensionNumbers(
    offset_dims=(), collapsed_slice_dims=(0,), start_index_map=(0,))


def _take16(v, perm):
    return lax.gather(v, perm[:, None], _DNUMS, slice_sizes=(1,),
                      mode=lax.GatherScatterMode.PROMISE_IN_BOUNDS)


def _allsum(v, lanes):
    for k in range(4):
        v = v + _take16(v, lanes ^ (1 << k))
    return v


def _rsqrt(s):
    bi = lax.bitcast_convert_type(s, jnp.int32)
    bi = jnp.int32(0x5F3759DF) - (bi >> 1)
    y = lax.bitcast_convert_type(bi, jnp.float32)
    half = jnp.float32(0.5) * s
    for _ in range(3):
        y = y * (jnp.float32(1.5) - half * y * y)
    return y


def _make_sc_call(B):
    info = plsc.get_sparse_core_info()
    NC, NS = info.num_cores, info.num_subcores
    NW = NC * NS
    b_per_w = B // NW
    n_quarters = b_per_w // QROWS
    groups_per_q = QROWS // LANES
    mesh = plsc.VectorSubcoreMesh(core_axis_name="c", subcore_axis_name="s")

    @functools.partial(
        pl.kernel,
        out_type=jax.ShapeDtypeStruct((B,), jnp.float32),
        mesh=mesh,
        scratch_types=[
            pltpu.VMEM((b_per_w,), jnp.int32),
            pltpu.VMEM((b_per_w,), jnp.int32),
            pltpu.VMEM((b_per_w,), jnp.int32),
            pltpu.VMEM((n_quarters, QROWS), jnp.int32),
            pltpu.VMEM((n_quarters, QROWS), jnp.int32),
            pltpu.VMEM((n_quarters, QROWS), jnp.int32),
            pltpu.VMEM((QROWS, 2 * D), jnp.float32),
            pltpu.VMEM((QROWS, 2 * D), jnp.float32),
            pltpu.VMEM((QROWS, 2 * D), jnp.float32),
            pltpu.VMEM((QROWS, 2 * D), jnp.float32),
            pltpu.VMEM((QROWS, 2 * D), jnp.float32),
            pltpu.VMEM((b_per_w,), jnp.float32),
            pltpu.SemaphoreType.DMA,
        ],
    )
    def sc_call(h_hbm, r_hbm, t_hbm, ent_lo_hbm, ent_hi_hbm, rel_hbm,
                out_hbm, idx_h, idx_r, idx_t, row4_h, row2_r, row4_t,
                h_lo, h_hi, t_lo, t_hi, r_buf, out_scr, sem):
        wid = lax.axis_index("s") * NC + lax.axis_index("c")
        base = wid * b_per_w
        lanes = lax.iota(jnp.int32, LANES)

        c1 = pltpu.async_copy(h_hbm.at[pl.ds(base, b_per_w)], idx_h, sem)
        c2 = pltpu.async_copy(r_hbm.at[pl.ds(base, b_per_w)], idx_r, sem)
        c3 = pltpu.async_copy(t_hbm.at[pl.ds(base, b_per_w)], idx_t, sem)
        c1.wait()
        c2.wait()
        c3.wait()

        for q in range(n_quarters):
            for k in range(QROWS // LANES):
                sl = pl.ds(q * QROWS + k * LANES, LANES)
                dsl = pl.ds(k * LANES, LANES)
                row4_h[q, dsl] = idx_h[sl] >> 2
                row2_r[q, dsl] = idx_r[sl] >> 1
                row4_t[q, dsl] = idx_t[sl] >> 2

        def quarter_body(q, carry):
            g1 = pltpu.async_copy(ent_lo_hbm.at[row4_h.at[q]], h_lo, sem)
            g2 = pltpu.async_copy(ent_hi_hbm.at[row4_h.at[q]], h_hi, sem)
            g3 = pltpu.async_copy(ent_lo_hbm.at[row4_t.at[q]], t_lo, sem)
            g4 = pltpu.async_copy(ent_hi_hbm.at[row4_t.at[q]], t_hi, sem)
            g5 = pltpu.async_copy(rel_hbm.at[row2_r.at[q]], r_buf, sem)
            for g in (g1, g2, g3, g4, g5):
                g.wait()

            def group_body(g, c):
                he = idx_h[pl.ds(q * QROWS + g * LANES, LANES)]
                re = idx_r[pl.ds(q * QROWS + g * LANES, LANES)]
                te = idx_t[pl.ds(q * QROWS + g * LANES, LANES)]
                acc = jnp.zeros((LANES,), jnp.float32)
                for j in range(LANES):
                    i = g * LANES + j
                    hoff = (he[j] & 3) * HALF
                    toff = (te[j] & 3) * HALF
                    roff = (re[j] & 1) * D
                    hv = ([h_lo[i, pl.ds(hoff + kk * LANES, LANES)]
                           for kk in range(2)] +
                          [h_hi[i, pl.ds(hoff + kk * LANES, LANES)]
                           for kk in range(2)])
                    tv = ([t_lo[i, pl.ds(toff + kk * LANES, LANES)]
                           for kk in range(2)] +
                          [t_hi[i, pl.ds(toff + kk * LANES, LANES)]
                           for kk in range(2)])
                    rv = [r_buf[i, pl.ds(roff + kk * LANES, LANES)]
                          for kk in range(4)]
                    sh = _allsum(sum(v * v for v in hv), lanes)
                    sr = _allsum(sum(v * v for v in rv), lanes)
                    st = _allsum(sum(v * v for v in tv), lanes)
                    ih, ir, it = _rsqrt(sh), _rsqrt(sr), _rsqrt(st)
                    parts = [jnp.abs(a * ih + b * ir - d * it)
                             for a, b, d in zip(hv, rv, tv)]
                    sc = _allsum(parts[0] + parts[1] + parts[2] + parts[3],
                                 lanes)
                    acc = jnp.where(lanes == j, sc, acc)
                out_scr[pl.ds(q * QROWS + g * LANES, LANES)] = acc
                return c

            lax.fori_loop(0, groups_per_q, group_body, 0)
            return carry

        lax.fori_loop(0, n_quarters, quarter_body, 0)
        pltpu.sync_copy(out_scr, out_hbm.at[pl.ds(base, b_per_w)])

    return sc_call


def kernel(h, r, t, ent_table, rel_table):
    B = h.shape[0]
    V = ent_table.shape[0]
    VR = rel_table.shape[0]
    ent_lo = ent_table[:, :HALF].reshape(V // 4, 4 * HALF)
    ent_hi = ent_table[:, HALF:].reshape(V // 4, 4 * HALF)
    rel2 = rel_table.reshape(VR // 2, 2 * D)
    sc_call = _make_sc_call(B)
    score = sc_call(h.astype(jnp.int32), r.astype(jnp.int32),
                    t.astype(jnp.int32), ent_lo, ent_hi, rel2)
    return score[:, None]


# speedup vs baseline: 2.0519x; 2.0519x over previous
import functools

import jax
import jax.numpy as jnp
from jax import lax
from jax.experimental import pallas as pl
from jax.experimental.pallas import tpu as pltpu
from jax.experimental.pallas import tpu_sc as plsc

D = 64
LANES = 16
CW = 512
NCHUNK = 61
RANGE = NCHUNK * CW
MCAP = 1040
WCAP = 144
V_ENT = 1000000
TAIL0 = 32 * RANGE
TAIL1 = TAIL0 + CW

_CP = pltpu.CompilerParams(needs_layout_passes=False)


def _rsqrt(s):
    bi = lax.bitcast_convert_type(s, jnp.int32)
    bi = jnp.int32(0x5F3759DF) - (bi >> 1)
    y = lax.bitcast_convert_type(bi, jnp.float32)
    half = jnp.float32(0.5) * s
    for _ in range(3):
        y = y * (jnp.float32(1.5) - half * y * y)
    return y


def _make_extract_call(B):
    info = plsc.get_sparse_core_info()
    NC = info.num_cores
    mesh = plsc.VectorSubcoreMesh(core_axis_name="c", subcore_axis_name="s")
    n_scan = B // LANES
    out_len = B * D + D

    IB = 2048

    @functools.partial(
        pl.kernel,
        out_type=(jax.ShapeDtypeStruct((out_len,), jnp.float32),
                  jax.ShapeDtypeStruct((out_len,), jnp.float32)),
        mesh=mesh,
        compiler_params=_CP,
        scratch_types=[
            pltpu.VMEM((IB,), jnp.int32),
            pltpu.VMEM((MCAP,), jnp.int32),
            pltpu.VMEM((MCAP,), jnp.int32),
            pltpu.VMEM((MCAP,), jnp.int32),
            pltpu.VMEM((MCAP,), jnp.int32),
            pltpu.VMEM((WCAP,), jnp.int32),
            pltpu.VMEM((WCAP,), jnp.int32),
            pltpu.VMEM((D, CW), jnp.float32),
            pltpu.VMEM((D, CW), jnp.float32),
            pltpu.VMEM((D, D), jnp.float32),
            pltpu.VMEM((LANES, D), jnp.float32),
            pltpu.SemaphoreType.DMA,
            pltpu.SemaphoreType.DMA,
            pltpu.SemaphoreType.DMA,
            pltpu.SemaphoreType.DMA,
        ],
    )
    def extract_call(h_hbm, t_hbm, entT_hbm, hx_hbm, tx_hbm,
                     idxbuf, mh_ent, mh_slot, mt_ent, mt_slot,
                     wl_col, wl_slot, cb0, cb1, tailbuf, rowbufs,
                     sem0, sem1, semr, semi):
        wid = lax.axis_index("s") * NC + lax.axis_index("c")
        lo = wid * RANGE
        hi = lo + RANGE
        lanes = lax.iota(jnp.int32, LANES)
        trash = jnp.int32(B)

        sent = jnp.full((LANES,), jnp.int32(0x7FFFFFFF))
        strash = jnp.full((LANES,), trash)

        def prefill(i, c):
            sl = pl.ds(i * LANES, LANES)
            mh_ent[sl] = sent
            mt_ent[sl] = sent
            mh_slot[sl] = strash
            mt_slot[sl] = strash
            return c
        lax.fori_loop(0, MCAP // LANES, prefill, 0)

        is0 = wid == 0
        is1 = wid == 1

        def member(ev):
            m = (ev >= lo) & (ev < hi)
            m = m | (is0 & (ev >= TAIL0) & (ev < TAIL1))
            m = m | (is1 & (ev >= TAIL1))
            return m

        def prescan(idx_hbm, ment, mslot):
            def blk_body(b, pos):
                pltpu.async_copy(
                    idx_hbm.at[pl.ds(b * IB, IB)], idxbuf, semi).wait()

                def body(i, p):
                    ev = idxbuf[pl.ds(i * LANES, LANES)]
                    m = member(ev)
                    slots = b * IB + i * LANES + lanes
                    plsc.store_compressed(ment.at[pl.ds(p, LANES)], ev,
                                          mask=m)
                    plsc.store_compressed(mslot.at[pl.ds(p, LANES)], slots,
                                          mask=m)
                    cnt = plsc.all_reduce_population_count(m)
                    return p + cnt[0]
                return lax.fori_loop(0, IB // LANES, body, pos)
            return lax.fori_loop(0, B // IB, blk_body, jnp.int32(0))

        nh = prescan(h_hbm, mh_ent, mh_slot)
        nt = prescan(t_hbm, mt_ent, mt_slot)
        nh_g = (nh + LANES - 1) // LANES
        nt_g = (nt + LANES - 1) // LANES

        def process_list(buf, cbase, cwidth, ment, mslot, n_g, out_ref):
            def wpre(i, c):
                sl = pl.ds(i * LANES, LANES)
                wl_col[sl] = jnp.zeros((LANES,), jnp.int32)
                wl_slot[sl] = strash
                return c
            lax.fori_loop(0, WCAP // LANES, wpre, 0)

            def scan_body(i, pos):
                ev = ment[pl.ds(i * LANES, LANES)]
                sv = mslot[pl.ds(i * LANES, LANES)]
                m = (ev >= cbase) & (ev < cbase + cwidth)
                cols = ev - cbase
                plsc.store_compressed(wl_col.at[pl.ds(pos, LANES)], cols,
                                      mask=m)
                plsc.store_compressed(wl_slot.at[pl.ds(pos, LANES)], sv,
                                      mask=m)
                cnt = plsc.all_reduce_population_count(m)
                return pos + cnt[0]
            nw = lax.fori_loop(0, n_g, scan_body, jnp.int32(0))

            def grp_body(g, c):
                cols16 = wl_col[pl.ds(g * LANES, LANES)] & (cwidth - 1)
                slots16 = wl_slot[pl.ds(g * LANES, LANES)]
                for m in range(LANES):
                    cm = jnp.full((LANES,), cols16[m])
                    for k in range(D // LANES):
                        v = plsc.load_gather(buf, [k * LANES + lanes, cm])
                        rowbufs[m, pl.ds(k * LANES, LANES)] = v
                descs = []
                for m in range(LANES):
                    descs.append(pltpu.async_copy(
                        rowbufs.at[m],
                        out_ref.at[pl.ds(slots16[m] * D, D)], semr))
                for d_ in descs:
                    d_.wait()
                return c
            lax.fori_loop(0, (nw + LANES - 1) // LANES, grp_body, 0)

        def process_chunk(buf, cbase, cwidth):
            process_list(buf, cbase, cwidth, mh_ent, mh_slot, nh_g, hx_hbm)
            process_list(buf, cbase, cwidth, mt_ent, mt_slot, nt_g, tx_hbm)

        def chunk_src(q):
            return entT_hbm.at[:, pl.ds(lo + q * CW, CW)]

        pltpu.async_copy(chunk_src(0), cb0, sem0)

        def pipe_body(qq, c):
            q0 = qq * 2
            pltpu.async_copy(chunk_src(q0 + 1), cb1, sem1)
            pltpu.make_async_copy(chunk_src(q0), cb0, sem0).wait()
            process_chunk(cb0, lo + q0 * CW, CW)
            pltpu.async_copy(chunk_src(q0 + 2), cb0, sem0)
            pltpu.make_async_copy(chunk_src(q0 + 1), cb1, sem1).wait()
            process_chunk(cb1, lo + (q0 + 1) * CW, CW)
            return c
        lax.fori_loop(0, (NCHUNK - 1) // 2, pipe_body, 0)
        pltpu.make_async_copy(chunk_src(NCHUNK - 1), cb0, sem0).wait()
        process_chunk(cb0, lo + (NCHUNK - 1) * CW, CW)

        @pl.when(is0)
        def _():
            pltpu.async_copy(
                entT_hbm.at[:, pl.ds(TAIL0, CW)], cb1, sem1).wait()
            process_chunk(cb1, jnp.int32(TAIL0), CW)

        @pl.when(is1)
        def _():
            pltpu.async_copy(
                entT_hbm.at[:, pl.ds(TAIL1, D)], tailbuf, sem1).wait()
            process_chunk(tailbuf, jnp.int32(TAIL1), D)

    return extract_call


def _make_score_call(B, VR):
    info = plsc.get_sparse_core_info()
    NC, NS = info.num_cores, info.num_subcores
    NW = NC * NS
    b_per_w = B // NW
    WAVE = b_per_w // 2
    mesh = plsc.VectorSubcoreMesh(core_axis_name="c", subcore_axis_name="s")
    wave_len = WAVE * D

    @functools.partial(
        pl.kernel,
        out_type=jax.ShapeDtypeStruct((B,), jnp.float32),
        mesh=mesh,
        compiler_params=_CP,
        scratch_types=[
            pltpu.VMEM((b_per_w,), jnp.int32),
            pltpu.VMEM((D, VR), jnp.float32),
            pltpu.VMEM((wave_len,), jnp.float32),
            pltpu.VMEM((wave_len,), jnp.float32),
            pltpu.VMEM((b_per_w,), jnp.float32),
            pltpu.SemaphoreType.DMA,
        ],
    )
    def score_call(r_hbm, relT_hbm, hx_hbm, tx_hbm, out_hbm,
                   ridx, relv, hflat, tflat, out_scr, sem):
        wid = lax.axis_index("s") * NC + lax.axis_index("c")
        base = wid * b_per_w
        lanes = lax.iota(jnp.int32, LANES)

        c1 = pltpu.async_copy(r_hbm.at[pl.ds(base, b_per_w)], ridx, sem)
        c2 = pltpu.async_copy(relT_hbm, relv, sem)
        c1.wait()
        c2.wait()

        zeros = jnp.zeros((LANES,), jnp.float32)

        for wave in range(2):
            wbase = base + wave * WAVE
            c3 = pltpu.async_copy(hx_hbm.at[pl.ds(wbase * D, wave_len)],
                                  hflat, sem)
            c4 = pltpu.async_copy(tx_hbm.at[pl.ds(wbase * D, wave_len)],
                                  tflat, sem)
            c3.wait()
            c4.wait()

            def group_body(g, c):
                fb = (g * LANES + lanes) * D
                re16 = ridx[pl.ds(wave * WAVE + g * LANES, LANES)]

                def sq_body(f, accs):
                    ah, ar, at_ = accs
                    hv = plsc.load_gather(hflat, [fb + f])
                    tv = plsc.load_gather(tflat, [fb + f])
                    rv = plsc.load_gather(relv, [jnp.full((LANES,), f), re16])
                    return ah + hv * hv, ar + rv * rv, at_ + tv * tv

                sh, sr, st = lax.fori_loop(0, D, sq_body,
                                           (zeros, zeros, zeros))
                ih, ir, it = _rsqrt(sh), _rsqrt(sr), _rsqrt(st)

                def sc_body(f, acc):
                    hv = plsc.load_gather(hflat, [fb + f])
                    tv = plsc.load_gather(tflat, [fb + f])
                    rv = plsc.load_gather(relv, [jnp.full((LANES,), f), re16])
                    return acc + jnp.abs(hv * ih + rv * ir - tv * it)

                sc = lax.fori_loop(0, D, sc_body, zeros)
                out_scr[pl.ds(wave * WAVE + g * LANES, LANES)] = sc
                return c

            lax.fori_loop(0, WAVE // LANES, group_body, 0)

        pltpu.sync_copy(out_scr, out_hbm.at[pl.ds(base, b_per_w)])

    return score_call


def kernel(h, r, t, ent_table, rel_table):
    B = h.shape[0]
    VR = rel_table.shape[0]
    entT = ent_table.T
    relT = rel_table.T
    extract_call = _make_extract_call(B)
    hx, tx = extract_call(h.astype(jnp.int32), t.astype(jnp.int32), entT)
    score_call = _make_score_call(B, VR)
    score = score_call(r.astype(jnp.int32), relT, hx, tx)
    return score[:, None]
